# Initial kernel scaffold; baseline (speedup 1.0000x reference)
import jax
import jax.numpy as jnp
from jax.experimental import pallas as pl


def kernel(x, edge_index, W1, b1, W2, b2):
    raise NotImplementedError("write your pallas kernel here")


# speedup vs baseline: 6.6646x; 6.6646x over previous
import functools

import jax
import jax.numpy as jnp
from jax import lax
from jax.experimental import pallas as pl
from jax.experimental.pallas import tpu as pltpu
from jax.experimental.pallas import tpu_sc as plsc

N = 10000
E = 320000
D = 128
DOUT = 40
D2 = 128
NPAD = 10240
NC = 2
NS = 16
CHUNK = 80
ROWS_PER_TILE = NPAD // NS

_MESH = dict(core_axis_name="c", subcore_axis_name="s")


def _sc_degrees(edges_flat):
    ept = E // NS
    nchunk = ept // CHUNK

    @functools.partial(
        pl.kernel,
        out_type=jax.ShapeDtypeStruct((NC * NPAD,), jnp.float32),
        mesh=plsc.VectorSubcoreMesh(**_MESH),
        scratch_types=[
            pltpu.VMEM((CHUNK,), jnp.int32),
            pltpu.VMEM((CHUNK,), jnp.float32),
            pltpu.VMEM((ROWS_PER_TILE,), jnp.float32),
            pltpu.VMEM_SHARED((NPAD,), jnp.float32),
        ],
    )
    def deg_kernel(edges_hbm, out_hbm, idx_v, ones_v, zero_v, deg_s):
        c = lax.axis_index("c")
        s = lax.axis_index("s")

        def fill_zero(i, carry):
            zero_v[pl.ds(i * 16, 16)] = jnp.zeros((16,), jnp.float32)
            return carry

        lax.fori_loop(0, ROWS_PER_TILE // 16, fill_zero, 0)

        def fill_one(i, carry):
            ones_v[pl.ds(i * 16, 16)] = jnp.ones((16,), jnp.float32)
            return carry

        lax.fori_loop(0, CHUNK // 16, fill_one, 0)

        pltpu.sync_copy(zero_v, deg_s.at[pl.ds(s * ROWS_PER_TILE, ROWS_PER_TILE)])
        plsc.subcore_barrier()

        base = c * E + s * ept

        def step(j, carry):
            off = pl.multiple_of(base + j * CHUNK, 8)
            pltpu.sync_copy(edges_hbm.at[pl.ds(off, CHUNK)], idx_v)
            pltpu.sync_copy(ones_v, deg_s.at[idx_v], add=True)
            return carry

        lax.fori_loop(0, nchunk, step, 0)
        plsc.subcore_barrier()
        pltpu.sync_copy(
            deg_s.at[pl.ds(s * ROWS_PER_TILE, ROWS_PER_TILE)],
            out_hbm.at[pl.ds(c * NPAD + s * ROWS_PER_TILE, ROWS_PER_TILE)],
        )

    return deg_kernel(edges_flat)


def _sc_scatter(edges_flat, feat, dfeat):
    e_per_core = E // NC
    ept = e_per_core // NS
    nchunk = ept // CHUNK

    @functools.partial(
        pl.kernel,
        out_type=jax.ShapeDtypeStruct((NC * NPAD, dfeat), jnp.float32),
        mesh=plsc.VectorSubcoreMesh(**_MESH),
        scratch_types=[
            pltpu.VMEM((CHUNK,), jnp.int32),
            pltpu.VMEM((CHUNK,), jnp.int32),
            pltpu.VMEM((CHUNK, dfeat), jnp.float32),
            pltpu.VMEM((CHUNK, dfeat), jnp.float32),
            pltpu.SemaphoreType.DMA,
            pltpu.VMEM_SHARED((NPAD, dfeat), jnp.float32),
        ],
    )
    def scat_kernel(edges_hbm, feat_hbm, out_hbm, sidx, didx, rows, zrows, sem, agg_s):
        c = lax.axis_index("c")
        s = lax.axis_index("s")
        vpr = dfeat // 16

        def fill_zero(k, carry):
            zrows[k // vpr, pl.ds((k % vpr) * 16, 16)] = jnp.zeros((16,), jnp.float32)
            return carry

        lax.fori_loop(0, CHUNK * vpr, fill_zero, 0)

        def zero_chunk(k, carry):
            pltpu.sync_copy(
                zrows, agg_s.at[pl.ds(s * ROWS_PER_TILE + k * CHUNK, CHUNK)]
            )
            return carry

        lax.fori_loop(0, ROWS_PER_TILE // CHUNK, zero_chunk, 0)
        plsc.subcore_barrier()

        base = c * e_per_core + s * ept

        def step(j, carry):
            off = pl.multiple_of(base + j * CHUNK, 8)
            pltpu.sync_copy(edges_hbm.at[pl.ds(off, CHUNK)], sidx)
            pltpu.sync_copy(edges_hbm.at[pl.ds(E + off, CHUNK)], didx)
            pltpu.async_copy(feat_hbm.at[sidx], rows, sem).wait()
            pltpu.sync_copy(rows, agg_s.at[didx], add=True)
            return carry

        lax.fori_loop(0, nchunk, step, 0)
        plsc.subcore_barrier()
        pltpu.sync_copy(
            agg_s.at[pl.ds(s * ROWS_PER_TILE, ROWS_PER_TILE)],
            out_hbm.at[pl.ds(c * NPAD + s * ROWS_PER_TILE, ROWS_PER_TILE)],
        )

    return scat_kernel(edges_flat, feat)


_TCR = 2000


def _norm_body(x_ref, do_ref, di_ref, xs_ref, ns_ref, nd_ref):
    ns = lax.rsqrt(do_ref[...] + 1.0)
    nd = lax.rsqrt(di_ref[...] + 1.0)
    xs_ref[...] = x_ref[...] * ns
    ns_ref[...] = ns
    nd_ref[...] = nd


def _tc_norm(x, do_col, di_col):
    row = lambda i: (i, 0)
    return pl.pallas_call(
        _norm_body,
        grid=(N // _TCR,),
        in_specs=[
            pl.BlockSpec((_TCR, D), row),
            pl.BlockSpec((_TCR, 1), row),
            pl.BlockSpec((_TCR, 1), row),
        ],
        out_specs=[
            pl.BlockSpec((_TCR, D), row),
            pl.BlockSpec((_TCR, 1), row),
            pl.BlockSpec((_TCR, 1), row),
        ],
        out_shape=[
            jax.ShapeDtypeStruct((N, D), jnp.float32),
            jax.ShapeDtypeStruct((N, 1), jnp.float32),
            jax.ShapeDtypeStruct((N, 1), jnp.float32),
        ],
    )(x, do_col, di_col)


def _mm_body(a0_ref, a1_ref, xs_ref, nd_ref, ns_ref, w1_ref, b1_ref, w2_ref, g2_ref):
    a = (a0_ref[...] + a1_ref[...] + xs_ref[...]) * nd_ref[...]
    h1 = (
        jnp.dot(a, w1_ref[...], preferred_element_type=jnp.float32,
                precision=lax.Precision.HIGHEST)
        + b1_ref[...]
    )
    g2_ref[...] = jnp.dot(
        h1 * ns_ref[...], w2_ref[...], preferred_element_type=jnp.float32,
        precision=lax.Precision.HIGHEST,
    )


def _tc_matmuls(a0, a1, xs, nd_col, ns_col, W1, b1r, W2p):
    row = lambda i: (i, 0)
    full = lambda i: (0, 0)
    return pl.pallas_call(
        _mm_body,
        grid=(N // _TCR,),
        in_specs=[
            pl.BlockSpec((_TCR, D), row),
            pl.BlockSpec((_TCR, D), row),
            pl.BlockSpec((_TCR, D), row),
            pl.BlockSpec((_TCR, 1), row),
            pl.BlockSpec((_TCR, 1), row),
            pl.BlockSpec((D, D), full),
            pl.BlockSpec((1, D), full),
            pl.BlockSpec((D, D2), full),
        ],
        out_specs=pl.BlockSpec((_TCR, D2), row),
        out_shape=jax.ShapeDtypeStruct((N, D2), jnp.float32),
    )(a0, a1, xs, nd_col, ns_col, W1, b1r, W2p)


def _final_body(a0_ref, a1_ref, g2_ref, nd_ref, b2_ref, out_ref):
    out_ref[...] = (
        (a0_ref[...] + a1_ref[...] + g2_ref[...]) * nd_ref[...] + b2_ref[...]
    )


def _tc_final(a0, a1, g2, nd_col, b2r):
    row = lambda i: (i, 0)
    full = lambda i: (0, 0)
    return pl.pallas_call(
        _final_body,
        grid=(N // _TCR,),
        in_specs=[
            pl.BlockSpec((_TCR, D2), row),
            pl.BlockSpec((_TCR, D2), row),
            pl.BlockSpec((_TCR, D2), row),
            pl.BlockSpec((_TCR, 1), row),
            pl.BlockSpec((1, D2), full),
        ],
        out_specs=pl.BlockSpec((_TCR, D2), row),
        out_shape=jax.ShapeDtypeStruct((N, D2), jnp.float32),
    )(a0, a1, g2, nd_col, b2r)


def kernel(x, edge_index, W1, b1, W2, b2):
    edges_flat = edge_index.reshape(2 * E)

    deg = _sc_degrees(edges_flat)
    do_col = deg[:N, None]
    di_col = deg[NPAD:NPAD + N, None]

    xs, ns_col, nd_col = _tc_norm(x, do_col, di_col)

    agg = _sc_scatter(edges_flat, xs, D)
    g2 = _tc_matmuls(
        agg[:N], agg[NPAD:NPAD + N], xs, nd_col, ns_col,
        W1, b1[None, :], jnp.pad(W2, ((0, 0), (0, D2 - DOUT))),
    )

    agg2 = _sc_scatter(edges_flat, g2, D2)
    out = _tc_final(
        agg2[:N], agg2[NPAD:NPAD + N], g2, nd_col,
        jnp.pad(b2, (0, D2 - DOUT))[None, :],
    )
    return out[:, :DOUT]


# speedup vs baseline: 17.2597x; 2.5898x over previous
import functools

import jax
import jax.numpy as jnp
from jax import lax
from jax.experimental import pallas as pl
from jax.experimental.pallas import tpu as pltpu
from jax.experimental.pallas import tpu_sc as plsc

N = 10000
E = 320000
D = 128
DOUT = 40
D2 = 128
NPAD = 10240
NC = 2
NS = 16
CHUNK = 80
ROWS_PER_TILE = NPAD // NS

_MESH = dict(core_axis_name="c", subcore_axis_name="s")


def _sc_degrees(edges3):
    ept = E // NS
    nchunk = ept // CHUNK

    @functools.partial(
        pl.kernel,
        out_type=jax.ShapeDtypeStruct((NC * NPAD,), jnp.float32),
        mesh=plsc.VectorSubcoreMesh(**_MESH),
        scratch_types=[
            pltpu.VMEM((nchunk, CHUNK), jnp.int32),
            pltpu.VMEM((CHUNK,), jnp.float32),
            pltpu.VMEM((ROWS_PER_TILE,), jnp.float32),
            pltpu.SemaphoreType.DMA,
            pltpu.VMEM_SHARED((NPAD,), jnp.float32),
        ],
    )
    def deg_kernel(edges_hbm, out_hbm, idx_all, ones_v, zero_v, ssem, deg_s):
        c = lax.axis_index("c")
        s = lax.axis_index("s")

        def fill_zero(i, carry):
            zero_v[pl.ds(i * 16, 16)] = jnp.zeros((16,), jnp.float32)
            return carry

        lax.fori_loop(0, ROWS_PER_TILE // 16, fill_zero, 0)

        def fill_one(i, carry):
            ones_v[pl.ds(i * 16, 16)] = jnp.ones((16,), jnp.float32)
            return carry

        lax.fori_loop(0, CHUNK // 16, fill_one, 0)

        pltpu.sync_copy(edges_hbm.at[c * NS + s], idx_all)
        pltpu.sync_copy(zero_v, deg_s.at[pl.ds(s * ROWS_PER_TILE, ROWS_PER_TILE)])
        plsc.subcore_barrier()

        def step(j, carry):
            pltpu.async_copy(ones_v, deg_s.at[idx_all.at[j]], ssem, add=True)
            return carry

        lax.fori_loop(0, nchunk, step, 0)

        def drain(j, carry):
            pltpu.make_async_copy(ones_v, deg_s.at[idx_all.at[0]], ssem).wait()
            return carry

        lax.fori_loop(0, nchunk, drain, 0)
        plsc.subcore_barrier()
        pltpu.sync_copy(
            deg_s.at[pl.ds(s * ROWS_PER_TILE, ROWS_PER_TILE)],
            out_hbm.at[pl.ds(c * NPAD + s * ROWS_PER_TILE, ROWS_PER_TILE)],
        )

    return deg_kernel(edges3)


def _sc_scatter(src3, dst3, feat, dfeat):
    e_per_core = E // NC
    ept = e_per_core // NS
    csz = 125
    nchunk = ept // csz
    NB = 2
    NI = 4
    ZR = 16

    @functools.partial(
        pl.kernel,
        out_type=jax.ShapeDtypeStruct((NC * NPAD, dfeat), jnp.float32),
        mesh=plsc.VectorSubcoreMesh(**_MESH),
        scratch_types=[
            pltpu.VMEM((NI, csz), jnp.int32),
            pltpu.VMEM((NI, csz), jnp.int32),
            pltpu.VMEM((NB, csz, dfeat), jnp.float32),
            pltpu.VMEM((ZR, dfeat), jnp.float32),
            pltpu.SemaphoreType.DMA,
            pltpu.SemaphoreType.DMA,
            pltpu.SemaphoreType.DMA,
            pltpu.VMEM_SHARED((NPAD, dfeat), jnp.float32),
        ],
    )
    def scat_kernel(src_hbm, dst_hbm, feat_hbm, out_hbm, sidx, didx,
                    rows, zrows, isem, gsem, ssem, agg_s):
        c = lax.axis_index("c")
        s = lax.axis_index("s")
        wid = c * NS + s
        vpr = dfeat // 16

        def fill_zero(k, carry):
            zrows[k // vpr, pl.ds((k % vpr) * 16, 16)] = jnp.zeros((16,), jnp.float32)
            return carry

        lax.fori_loop(0, ZR * vpr, fill_zero, 0)

        def zero_chunk(k, carry):
            pltpu.sync_copy(
                zrows, agg_s.at[pl.ds(s * ROWS_PER_TILE + k * ZR, ZR)]
            )
            return carry

        lax.fori_loop(0, ROWS_PER_TILE // ZR, zero_chunk, 0)
        plsc.subcore_barrier()

        def load_idx(j, slot):
            pltpu.async_copy(src_hbm.at[wid, j], sidx.at[slot], isem)
            pltpu.async_copy(dst_hbm.at[wid, j], didx.at[slot], isem)

        def wait_idx():
            pltpu.make_async_copy(src_hbm.at[wid, 0], sidx.at[0], isem).wait()
            pltpu.make_async_copy(dst_hbm.at[wid, 0], didx.at[0], isem).wait()

        def gath(b, slot):
            pltpu.async_copy(feat_hbm.at[sidx.at[slot]], rows.at[b], gsem)

        def wait_gath():
            pltpu.make_async_copy(
                feat_hbm.at[sidx.at[0]], rows.at[0], gsem
            ).wait()

        def scat(b, slot):
            pltpu.async_copy(rows.at[b], agg_s.at[didx.at[slot]], ssem, add=True)

        def wait_scat():
            pltpu.make_async_copy(
                rows.at[0], agg_s.at[didx.at[0]], ssem
            ).wait()

        for j in range(NI):
            load_idx(j, j)
        wait_idx()
        wait_idx()
        gath(0, 0)
        gath(1, 1)
        wait_gath()
        scat(0, 0)

        def step(k, carry):
            wait_scat()
            load_idx(k + 3, lax.rem(k + 3, NI))
            wait_idx()
            gath(lax.rem(k + 1, NB), lax.rem(k + 1, NI))
            wait_gath()
            scat(lax.rem(k, NB), lax.rem(k, NI))
            return carry

        lax.fori_loop(1, nchunk - 3, step, 0)

        for k in (nchunk - 3, nchunk - 2):
            wait_scat()
            wait_idx()
            gath((k + 1) % NB, (k + 1) % NI)
            wait_gath()
            scat(k % NB, k % NI)
        wait_scat()
        wait_gath()
        scat((nchunk - 1) % NB, (nchunk - 1) % NI)
        wait_scat()

        plsc.subcore_barrier()
        pltpu.sync_copy(
            agg_s.at[pl.ds(s * ROWS_PER_TILE, ROWS_PER_TILE)],
            out_hbm.at[pl.ds(c * NPAD + s * ROWS_PER_TILE, ROWS_PER_TILE)],
        )

    return scat_kernel(src3, dst3, feat)


_TCR = 2000


def _norm_body(x_ref, do_ref, di_ref, xs_ref, ns_ref, nd_ref):
    ns = lax.rsqrt(do_ref[...] + 1.0)
    nd = lax.rsqrt(di_ref[...] + 1.0)
    xs_ref[...] = x_ref[...] * ns
    ns_ref[...] = ns
    nd_ref[...] = nd


def _tc_norm(x, do_col, di_col):
    row = lambda i: (i, 0)
    return pl.pallas_call(
        _norm_body,
        grid=(N // _TCR,),
        in_specs=[
            pl.BlockSpec((_TCR, D), row),
            pl.BlockSpec((_TCR, 1), row),
            pl.BlockSpec((_TCR, 1), row),
        ],
        out_specs=[
            pl.BlockSpec((_TCR, D), row),
            pl.BlockSpec((_TCR, 1), row),
            pl.BlockSpec((_TCR, 1), row),
        ],
        out_shape=[
            jax.ShapeDtypeStruct((N, D), jnp.float32),
            jax.ShapeDtypeStruct((N, 1), jnp.float32),
            jax.ShapeDtypeStruct((N, 1), jnp.float32),
        ],
    )(x, do_col, di_col)


def _mm_body(a0_ref, a1_ref, xs_ref, nd_ref, ns_ref, w1_ref, b1_ref, w2_ref, g2_ref):
    a = (a0_ref[...] + a1_ref[...] + xs_ref[...]) * nd_ref[...]
    h1 = (
        jnp.dot(a, w1_ref[...], preferred_element_type=jnp.float32,
                precision=lax.Precision.HIGHEST)
        + b1_ref[...]
    )
    g2_ref[...] = jnp.dot(
        h1 * ns_ref[...], w2_ref[...], preferred_element_type=jnp.float32,
        precision=lax.Precision.HIGHEST,
    )


def _tc_matmuls(a0, a1, xs, nd_col, ns_col, W1, b1r, W2p):
    row = lambda i: (i, 0)
    full = lambda i: (0, 0)
    return pl.pallas_call(
        _mm_body,
        grid=(N // _TCR,),
        in_specs=[
            pl.BlockSpec((_TCR, D), row),
            pl.BlockSpec((_TCR, D), row),
            pl.BlockSpec((_TCR, D), row),
            pl.BlockSpec((_TCR, 1), row),
            pl.BlockSpec((_TCR, 1), row),
            pl.BlockSpec((D, D), full),
            pl.BlockSpec((1, D), full),
            pl.BlockSpec((D, D2), full),
        ],
        out_specs=pl.BlockSpec((_TCR, D2), row),
        out_shape=jax.ShapeDtypeStruct((N, D2), jnp.float32),
    )(a0, a1, xs, nd_col, ns_col, W1, b1r, W2p)


def _final_body(a0_ref, a1_ref, g2_ref, nd_ref, b2_ref, out_ref):
    out_ref[...] = (
        (a0_ref[...] + a1_ref[...] + g2_ref[...]) * nd_ref[...] + b2_ref[...]
    )


def _tc_final(a0, a1, g2, nd_col, b2r):
    row = lambda i: (i, 0)
    full = lambda i: (0, 0)
    return pl.pallas_call(
        _final_body,
        grid=(N // _TCR,),
        in_specs=[
            pl.BlockSpec((_TCR, D2), row),
            pl.BlockSpec((_TCR, D2), row),
            pl.BlockSpec((_TCR, D2), row),
            pl.BlockSpec((_TCR, 1), row),
            pl.BlockSpec((1, D2), full),
        ],
        out_specs=pl.BlockSpec((_TCR, D2), row),
        out_shape=jax.ShapeDtypeStruct((N, D2), jnp.float32),
    )(a0, a1, g2, nd_col, b2r)


def kernel(x, edge_index, W1, b1, W2, b2):
    ept_deg = E // NS
    ept_sc = E // (NC * NS)
    edges3 = edge_index.reshape(NC * NS, ept_deg // CHUNK, CHUNK)
    src3 = edge_index[0].reshape(NC * NS, ept_sc // 125, 125)
    dst3 = edge_index[1].reshape(NC * NS, ept_sc // 125, 125)

    deg = _sc_degrees(edges3)
    do_col = deg[:N, None]
    di_col = deg[NPAD:NPAD + N, None]

    xs, ns_col, nd_col = _tc_norm(x, do_col, di_col)

    agg = _sc_scatter(src3, dst3, xs, D)
    g2 = _tc_matmuls(
        agg[:N], agg[NPAD:NPAD + N], xs, nd_col, ns_col,
        W1, b1[None, :], jnp.pad(W2, ((0, 0), (0, D2 - DOUT))),
    )

    agg2 = _sc_scatter(src3, dst3, g2, D2)
    out = _tc_final(
        agg2[:N], agg2[NPAD:NPAD + N], g2, nd_col,
        jnp.pad(b2, (0, D2 - DOUT))[None, :],
    )
    return out[:, :DOUT]


# speedup vs baseline: 18.6650x; 1.0814x over previous
import functools

import jax
import jax.numpy as jnp
from jax import lax
from jax.experimental import pallas as pl
from jax.experimental.pallas import tpu as pltpu
from jax.experimental.pallas import tpu_sc as plsc

N = 10000
E = 320000
D = 128
DOUT = 40
D2 = 64
NPAD = 10240
NC = 2
NS = 16
CHUNK = 80
ROWS_PER_TILE = NPAD // NS

_MESH = dict(core_axis_name="c", subcore_axis_name="s")


def _sc_degrees(edges3):
    ept = E // NS
    nchunk = ept // CHUNK

    @functools.partial(
        pl.kernel,
        out_type=jax.ShapeDtypeStruct((NC * NPAD,), jnp.float32),
        mesh=plsc.VectorSubcoreMesh(**_MESH),
        scratch_types=[
            pltpu.VMEM((nchunk, CHUNK), jnp.int32),
            pltpu.VMEM((CHUNK,), jnp.float32),
            pltpu.VMEM((ROWS_PER_TILE,), jnp.float32),
            pltpu.SemaphoreType.DMA,
            pltpu.VMEM_SHARED((NPAD,), jnp.float32),
        ],
    )
    def deg_kernel(edges_hbm, out_hbm, idx_all, ones_v, zero_v, ssem, deg_s):
        c = lax.axis_index("c")
        s = lax.axis_index("s")

        def fill_zero(i, carry):
            zero_v[pl.ds(i * 16, 16)] = jnp.zeros((16,), jnp.float32)
            return carry

        lax.fori_loop(0, ROWS_PER_TILE // 16, fill_zero, 0)

        def fill_one(i, carry):
            ones_v[pl.ds(i * 16, 16)] = jnp.ones((16,), jnp.float32)
            return carry

        lax.fori_loop(0, CHUNK // 16, fill_one, 0)

        pltpu.sync_copy(edges_hbm.at[c * NS + s], idx_all)
        pltpu.sync_copy(zero_v, deg_s.at[pl.ds(s * ROWS_PER_TILE, ROWS_PER_TILE)])
        plsc.subcore_barrier()

        def step(j, carry):
            pltpu.async_copy(ones_v, deg_s.at[idx_all.at[j]], ssem, add=True)
            return carry

        lax.fori_loop(0, nchunk, step, 0)

        def drain(j, carry):
            pltpu.make_async_copy(ones_v, deg_s.at[idx_all.at[0]], ssem).wait()
            return carry

        lax.fori_loop(0, nchunk, drain, 0)
        plsc.subcore_barrier()
        pltpu.sync_copy(
            deg_s.at[pl.ds(s * ROWS_PER_TILE, ROWS_PER_TILE)],
            out_hbm.at[pl.ds(c * NPAD + s * ROWS_PER_TILE, ROWS_PER_TILE)],
        )

    return deg_kernel(edges3)


def _sc_scatter(src3, dst3, feat, dfeat):
    e_per_core = E // NC
    ept = e_per_core // NS
    csz = 125
    nchunk = ept // csz
    NB = 2
    NI = 4
    ZR = 16

    @functools.partial(
        pl.kernel,
        out_type=jax.ShapeDtypeStruct((NC * NPAD, dfeat), jnp.float32),
        mesh=plsc.VectorSubcoreMesh(**_MESH),
        compiler_params=pltpu.CompilerParams(
            use_tc_tiling_on_sc=(dfeat % 128 == 0)
        ),
        scratch_types=[
            pltpu.VMEM((NI, csz), jnp.int32),
            pltpu.VMEM((NI, csz), jnp.int32),
            pltpu.VMEM((NB, csz, dfeat), jnp.float32),
            pltpu.VMEM((ZR, dfeat), jnp.float32),
            pltpu.SemaphoreType.DMA,
            pltpu.SemaphoreType.DMA,
            pltpu.SemaphoreType.DMA,
            pltpu.VMEM_SHARED((NPAD, dfeat), jnp.float32),
        ],
    )
    def scat_kernel(src_hbm, dst_hbm, feat_hbm, out_hbm, sidx, didx,
                    rows, zrows, isem, gsem, ssem, agg_s):
        c = lax.axis_index("c")
        s = lax.axis_index("s")
        wid = c * NS + s
        vpr = dfeat // 16

        def fill_zero(k, carry):
            zrows[k // vpr, pl.ds((k % vpr) * 16, 16)] = jnp.zeros((16,), jnp.float32)
            return carry

        lax.fori_loop(0, ZR * vpr, fill_zero, 0)

        def zero_chunk(k, carry):
            pltpu.sync_copy(
                zrows, agg_s.at[pl.ds(s * ROWS_PER_TILE + k * ZR, ZR)]
            )
            return carry

        lax.fori_loop(0, ROWS_PER_TILE // ZR, zero_chunk, 0)
        plsc.subcore_barrier()

        def load_idx(j, slot):
            pltpu.async_copy(src_hbm.at[wid, j], sidx.at[slot], isem)
            pltpu.async_copy(dst_hbm.at[wid, j], didx.at[slot], isem)

        def wait_idx():
            pltpu.make_async_copy(src_hbm.at[wid, 0], sidx.at[0], isem).wait()
            pltpu.make_async_copy(dst_hbm.at[wid, 0], didx.at[0], isem).wait()

        def gath(b, slot):
            pltpu.async_copy(feat_hbm.at[sidx.at[slot]], rows.at[b], gsem)

        def wait_gath():
            pltpu.make_async_copy(
                feat_hbm.at[sidx.at[0]], rows.at[0], gsem
            ).wait()

        def scat(b, slot):
            pltpu.async_copy(rows.at[b], agg_s.at[didx.at[slot]], ssem, add=True)

        def wait_scat():
            pltpu.make_async_copy(
                rows.at[0], agg_s.at[didx.at[0]], ssem
            ).wait()

        for j in range(NI):
            load_idx(j, j)
        wait_idx()
        wait_idx()
        gath(0, 0)
        gath(1, 1)
        wait_gath()
        scat(0, 0)

        def step(k, carry):
            wait_scat()
            load_idx(k + 3, lax.rem(k + 3, NI))
            wait_idx()
            gath(lax.rem(k + 1, NB), lax.rem(k + 1, NI))
            wait_gath()
            scat(lax.rem(k, NB), lax.rem(k, NI))
            return carry

        lax.fori_loop(1, nchunk - 3, step, 0)

        for k in (nchunk - 3, nchunk - 2):
            wait_scat()
            wait_idx()
            gath((k + 1) % NB, (k + 1) % NI)
            wait_gath()
            scat(k % NB, k % NI)
        wait_scat()
        wait_gath()
        scat((nchunk - 1) % NB, (nchunk - 1) % NI)
        wait_scat()

        plsc.subcore_barrier()
        pltpu.sync_copy(
            agg_s.at[pl.ds(s * ROWS_PER_TILE, ROWS_PER_TILE)],
            out_hbm.at[pl.ds(c * NPAD + s * ROWS_PER_TILE, ROWS_PER_TILE)],
        )

    return scat_kernel(src3, dst3, feat)


_TCR = 2000


def _norm_body(x_ref, do_ref, di_ref, xs_ref, ns_ref, nd_ref):
    ns = lax.rsqrt(do_ref[...] + 1.0)
    nd = lax.rsqrt(di_ref[...] + 1.0)
    xs_ref[...] = x_ref[...] * ns
    ns_ref[...] = ns
    nd_ref[...] = nd


def _tc_norm(x, do_col, di_col):
    row = lambda i: (i, 0)
    return pl.pallas_call(
        _norm_body,
        grid=(N // _TCR,),
        in_specs=[
            pl.BlockSpec((_TCR, D), row),
            pl.BlockSpec((_TCR, 1), row),
            pl.BlockSpec((_TCR, 1), row),
        ],
        out_specs=[
            pl.BlockSpec((_TCR, D), row),
            pl.BlockSpec((_TCR, 1), row),
            pl.BlockSpec((_TCR, 1), row),
        ],
        out_shape=[
            jax.ShapeDtypeStruct((N, D), jnp.float32),
            jax.ShapeDtypeStruct((N, 1), jnp.float32),
            jax.ShapeDtypeStruct((N, 1), jnp.float32),
        ],
    )(x, do_col, di_col)


def _mm_body(a0_ref, a1_ref, xs_ref, nd_ref, ns_ref, w1_ref, b1_ref, w2_ref, g2_ref):
    a = (a0_ref[...] + a1_ref[...] + xs_ref[...]) * nd_ref[...]
    h1 = (
        jnp.dot(a, w1_ref[...], preferred_element_type=jnp.float32,
                precision=lax.Precision.HIGHEST)
        + b1_ref[...]
    )
    g2_ref[...] = jnp.dot(
        h1 * ns_ref[...], w2_ref[...], preferred_element_type=jnp.float32,
        precision=lax.Precision.HIGHEST,
    )


def _tc_matmuls(a0, a1, xs, nd_col, ns_col, W1, b1r, W2p):
    row = lambda i: (i, 0)
    full = lambda i: (0, 0)
    return pl.pallas_call(
        _mm_body,
        grid=(N // _TCR,),
        in_specs=[
            pl.BlockSpec((_TCR, D), row),
            pl.BlockSpec((_TCR, D), row),
            pl.BlockSpec((_TCR, D), row),
            pl.BlockSpec((_TCR, 1), row),
            pl.BlockSpec((_TCR, 1), row),
            pl.BlockSpec((D, D), full),
            pl.BlockSpec((1, D), full),
            pl.BlockSpec((D, D2), full),
        ],
        out_specs=pl.BlockSpec((_TCR, D2), row),
        out_shape=jax.ShapeDtypeStruct((N, D2), jnp.float32),
    )(a0, a1, xs, nd_col, ns_col, W1, b1r, W2p)


def _final_body(a0_ref, a1_ref, g2_ref, nd_ref, b2_ref, out_ref):
    out_ref[...] = (
        (a0_ref[...] + a1_ref[...] + g2_ref[...]) * nd_ref[...] + b2_ref[...]
    )


def _tc_final(a0, a1, g2, nd_col, b2r):
    row = lambda i: (i, 0)
    full = lambda i: (0, 0)
    return pl.pallas_call(
        _final_body,
        grid=(N // _TCR,),
        in_specs=[
            pl.BlockSpec((_TCR, D2), row),
            pl.BlockSpec((_TCR, D2), row),
            pl.BlockSpec((_TCR, D2), row),
            pl.BlockSpec((_TCR, 1), row),
            pl.BlockSpec((1, D2), full),
        ],
        out_specs=pl.BlockSpec((_TCR, D2), row),
        out_shape=jax.ShapeDtypeStruct((N, D2), jnp.float32),
    )(a0, a1, g2, nd_col, b2r)


def kernel(x, edge_index, W1, b1, W2, b2):
    ept_deg = E // NS
    ept_sc = E // (NC * NS)
    edges3 = edge_index.reshape(NC * NS, ept_deg // CHUNK, CHUNK)
    src3 = edge_index[0].reshape(NC * NS, ept_sc // 125, 125)
    dst3 = edge_index[1].reshape(NC * NS, ept_sc // 125, 125)

    deg = _sc_degrees(edges3)
    do_col = deg[:N, None]
    di_col = deg[NPAD:NPAD + N, None]

    xs, ns_col, nd_col = _tc_norm(x, do_col, di_col)

    agg = _sc_scatter(src3, dst3, xs, D)
    g2 = _tc_matmuls(
        agg[:N], agg[NPAD:NPAD + N], xs, nd_col, ns_col,
        W1, b1[None, :], jnp.pad(W2, ((0, 0), (0, D2 - DOUT))),
    )

    agg2 = _sc_scatter(src3, dst3, g2, D2)
    out = _tc_final(
        agg2[:N], agg2[NPAD:NPAD + N], g2, nd_col,
        jnp.pad(b2, (0, D2 - DOUT))[None, :],
    )
    return out[:, :DOUT]


# speedup vs baseline: 20.2719x; 1.0861x over previous
import functools

import jax
import jax.numpy as jnp
from jax import lax
from jax.experimental import pallas as pl
from jax.experimental.pallas import tpu as pltpu
from jax.experimental.pallas import tpu_sc as plsc

N = 10000
E = 320000
D = 128
DOUT = 40
D2 = 64
NPAD = 10240
NC = 2
NS = 16
CHUNK = 80
ROWS_PER_TILE = NPAD // NS

_MESH = dict(core_axis_name="c", subcore_axis_name="s")


def _sc_degrees(edges3):
    ept = E // NS
    nchunk = ept // CHUNK

    @functools.partial(
        pl.kernel,
        out_type=jax.ShapeDtypeStruct((NC * NPAD,), jnp.float32),
        mesh=plsc.VectorSubcoreMesh(**_MESH),
        scratch_types=[
            pltpu.VMEM((nchunk, CHUNK), jnp.int32),
            pltpu.VMEM((CHUNK,), jnp.float32),
            pltpu.VMEM((ROWS_PER_TILE,), jnp.float32),
            pltpu.SemaphoreType.DMA,
            pltpu.VMEM_SHARED((NPAD,), jnp.float32),
        ],
    )
    def deg_kernel(edges_hbm, out_hbm, idx_all, ones_v, zero_v, ssem, deg_s):
        c = lax.axis_index("c")
        s = lax.axis_index("s")

        def fill_zero(i, carry):
            zero_v[pl.ds(i * 16, 16)] = jnp.zeros((16,), jnp.float32)
            return carry

        lax.fori_loop(0, ROWS_PER_TILE // 16, fill_zero, 0)

        def fill_one(i, carry):
            ones_v[pl.ds(i * 16, 16)] = jnp.ones((16,), jnp.float32)
            return carry

        lax.fori_loop(0, CHUNK // 16, fill_one, 0)

        pltpu.sync_copy(edges_hbm.at[c * NS + s], idx_all)
        pltpu.sync_copy(zero_v, deg_s.at[pl.ds(s * ROWS_PER_TILE, ROWS_PER_TILE)])
        plsc.subcore_barrier()

        def step(j, carry):
            pltpu.async_copy(ones_v, deg_s.at[idx_all.at[j]], ssem, add=True)
            return carry

        lax.fori_loop(0, nchunk, step, 0)

        def drain(j, carry):
            pltpu.make_async_copy(ones_v, deg_s.at[idx_all.at[0]], ssem).wait()
            return carry

        lax.fori_loop(0, nchunk, drain, 0)
        plsc.subcore_barrier()
        pltpu.sync_copy(
            deg_s.at[pl.ds(s * ROWS_PER_TILE, ROWS_PER_TILE)],
            out_hbm.at[pl.ds(c * NPAD + s * ROWS_PER_TILE, ROWS_PER_TILE)],
        )

    return deg_kernel(edges3)


def _sc_scatter(src_flat, dst_flat, feat, dfeat):
    e_per_core = E // NC
    ept = e_per_core // NS
    csz = 80 if dfeat > 64 else 125
    NB = 3 if dfeat > 64 else 4
    nchunk = ept // csz
    NI = NB + 2
    ZR = 16
    src3 = src_flat.reshape(NC * NS, nchunk, csz)
    dst3 = dst_flat.reshape(NC * NS, nchunk, csz)

    @functools.partial(
        pl.kernel,
        out_type=jax.ShapeDtypeStruct((NC * NPAD, dfeat), jnp.float32),
        mesh=plsc.VectorSubcoreMesh(**_MESH),
        compiler_params=pltpu.CompilerParams(
            use_tc_tiling_on_sc=(dfeat % 128 == 0)
        ),
        scratch_types=[
            pltpu.VMEM((NI, csz), jnp.int32),
            pltpu.VMEM((NI, csz), jnp.int32),
            pltpu.VMEM((NB, csz, dfeat), jnp.float32),
            pltpu.VMEM((ZR, dfeat), jnp.float32),
            pltpu.SemaphoreType.DMA,
            pltpu.SemaphoreType.DMA,
            pltpu.SemaphoreType.DMA,
            pltpu.VMEM_SHARED((NPAD, dfeat), jnp.float32),
        ],
    )
    def scat_kernel(src_hbm, dst_hbm, feat_hbm, out_hbm, sidx, didx,
                    rows, zrows, isem, gsem, ssem, agg_s):
        c = lax.axis_index("c")
        s = lax.axis_index("s")
        wid = c * NS + s
        vpr = dfeat // 16

        def fill_zero(k, carry):
            zrows[k // vpr, pl.ds((k % vpr) * 16, 16)] = jnp.zeros((16,), jnp.float32)
            return carry

        lax.fori_loop(0, ZR * vpr, fill_zero, 0)

        def zero_chunk(k, carry):
            pltpu.sync_copy(
                zrows, agg_s.at[pl.ds(s * ROWS_PER_TILE + k * ZR, ZR)]
            )
            return carry

        lax.fori_loop(0, ROWS_PER_TILE // ZR, zero_chunk, 0)
        plsc.subcore_barrier()

        def load_idx(j, slot):
            pltpu.async_copy(src_hbm.at[wid, j], sidx.at[slot], isem)
            pltpu.async_copy(dst_hbm.at[wid, j], didx.at[slot], isem)

        def wait_idx():
            pltpu.make_async_copy(src_hbm.at[wid, 0], sidx.at[0], isem).wait()
            pltpu.make_async_copy(dst_hbm.at[wid, 0], didx.at[0], isem).wait()

        def gath(b, slot):
            pltpu.async_copy(feat_hbm.at[sidx.at[slot]], rows.at[b], gsem)

        def wait_gath():
            pltpu.make_async_copy(
                feat_hbm.at[sidx.at[0]], rows.at[0], gsem
            ).wait()

        def scat(b, slot):
            pltpu.async_copy(rows.at[b], agg_s.at[didx.at[slot]], ssem, add=True)

        def wait_scat():
            pltpu.make_async_copy(
                rows.at[0], agg_s.at[didx.at[0]], ssem
            ).wait()

        n = nchunk
        for j in range(NI):
            load_idx(j, j)
        for j in range(NB - 1):
            wait_idx()
            gath(j, j)
        wait_gath()
        scat(0, 0)
        wait_idx()
        gath((NB - 1) % NB, (NB - 1) % NI)
        wait_gath()
        scat(1 % NB, 1 % NI)

        def step(k, carry):
            wait_scat()
            load_idx(k + NI - 2, lax.rem(k + NI - 2, NI))
            wait_idx()
            gath(lax.rem(k + NB - 2, NB), lax.rem(k + NB - 2, NI))
            wait_gath()
            scat(lax.rem(k, NB), lax.rem(k, NI))
            return carry

        lax.fori_loop(2, n - NI + 2, step, 0)

        for k in range(n - NI + 2, n - NB + 2):
            wait_scat()
            wait_idx()
            gath((k + NB - 2) % NB, (k + NB - 2) % NI)
            wait_gath()
            scat(k % NB, k % NI)
        for k in range(n - NB + 2, n):
            wait_scat()
            wait_gath()
            scat(k % NB, k % NI)
        wait_scat()
        wait_scat()

        plsc.subcore_barrier()
        pltpu.sync_copy(
            agg_s.at[pl.ds(s * ROWS_PER_TILE, ROWS_PER_TILE)],
            out_hbm.at[pl.ds(c * NPAD + s * ROWS_PER_TILE, ROWS_PER_TILE)],
        )

    return scat_kernel(src3, dst3, feat)


_TCR = 2000


def _norm_body(x_ref, do_ref, di_ref, xs_ref, ns_ref, nd_ref):
    ns = lax.rsqrt(do_ref[...] + 1.0)
    nd = lax.rsqrt(di_ref[...] + 1.0)
    xs_ref[...] = x_ref[...] * ns
    ns_ref[...] = ns
    nd_ref[...] = nd


def _tc_norm(x, do_col, di_col):
    row = lambda i: (i, 0)
    return pl.pallas_call(
        _norm_body,
        grid=(N // _TCR,),
        in_specs=[
            pl.BlockSpec((_TCR, D), row),
            pl.BlockSpec((_TCR, 1), row),
            pl.BlockSpec((_TCR, 1), row),
        ],
        out_specs=[
            pl.BlockSpec((_TCR, D), row),
            pl.BlockSpec((_TCR, 1), row),
            pl.BlockSpec((_TCR, 1), row),
        ],
        out_shape=[
            jax.ShapeDtypeStruct((N, D), jnp.float32),
            jax.ShapeDtypeStruct((N, 1), jnp.float32),
            jax.ShapeDtypeStruct((N, 1), jnp.float32),
        ],
    )(x, do_col, di_col)


def _mm_body(a0_ref, a1_ref, xs_ref, nd_ref, ns_ref, w1_ref, b1_ref, w2_ref, g2_ref):
    a = (a0_ref[...] + a1_ref[...] + xs_ref[...]) * nd_ref[...]
    h1 = (
        jnp.dot(a, w1_ref[...], preferred_element_type=jnp.float32,
                precision=lax.Precision.HIGHEST)
        + b1_ref[...]
    )
    g2_ref[...] = jnp.dot(
        h1 * ns_ref[...], w2_ref[...], preferred_element_type=jnp.float32,
        precision=lax.Precision.HIGHEST,
    )


def _tc_matmuls(a0, a1, xs, nd_col, ns_col, W1, b1r, W2p):
    row = lambda i: (i, 0)
    full = lambda i: (0, 0)
    return pl.pallas_call(
        _mm_body,
        grid=(N // _TCR,),
        in_specs=[
            pl.BlockSpec((_TCR, D), row),
            pl.BlockSpec((_TCR, D), row),
            pl.BlockSpec((_TCR, D), row),
            pl.BlockSpec((_TCR, 1), row),
            pl.BlockSpec((_TCR, 1), row),
            pl.BlockSpec((D, D), full),
            pl.BlockSpec((1, D), full),
            pl.BlockSpec((D, D2), full),
        ],
        out_specs=pl.BlockSpec((_TCR, D2), row),
        out_shape=jax.ShapeDtypeStruct((N, D2), jnp.float32),
    )(a0, a1, xs, nd_col, ns_col, W1, b1r, W2p)


def _final_body(a0_ref, a1_ref, g2_ref, nd_ref, b2_ref, out_ref):
    out_ref[...] = (
        (a0_ref[...] + a1_ref[...] + g2_ref[...]) * nd_ref[...] + b2_ref[...]
    )


def _tc_final(a0, a1, g2, nd_col, b2r):
    row = lambda i: (i, 0)
    full = lambda i: (0, 0)
    return pl.pallas_call(
        _final_body,
        grid=(N // _TCR,),
        in_specs=[
            pl.BlockSpec((_TCR, D2), row),
            pl.BlockSpec((_TCR, D2), row),
            pl.BlockSpec((_TCR, D2), row),
            pl.BlockSpec((_TCR, 1), row),
            pl.BlockSpec((1, D2), full),
        ],
        out_specs=pl.BlockSpec((_TCR, D2), row),
        out_shape=jax.ShapeDtypeStruct((N, D2), jnp.float32),
    )(a0, a1, g2, nd_col, b2r)


def kernel(x, edge_index, W1, b1, W2, b2):
    ept_deg = E // NS
    edges3 = edge_index.reshape(NC * NS, ept_deg // CHUNK, CHUNK)
    src_flat = edge_index[0]
    dst_flat = edge_index[1]

    deg = _sc_degrees(edges3)
    do_col = deg[:N, None]
    di_col = deg[NPAD:NPAD + N, None]

    xs, ns_col, nd_col = _tc_norm(x, do_col, di_col)

    agg = _sc_scatter(src_flat, dst_flat, xs, D)
    g2 = _tc_matmuls(
        agg[:N], agg[NPAD:NPAD + N], xs, nd_col, ns_col,
        W1, b1[None, :], jnp.pad(W2, ((0, 0), (0, D2 - DOUT))),
    )

    agg2 = _sc_scatter(src_flat, dst_flat, g2, D2)
    out = _tc_final(
        agg2[:N], agg2[NPAD:NPAD + N], g2, nd_col,
        jnp.pad(b2, (0, D2 - DOUT))[None, :],
    )
    return out[:, :DOUT]


# speedup vs baseline: 20.3638x; 1.0045x over previous
import functools

import jax
import jax.numpy as jnp
from jax import lax
from jax.experimental import pallas as pl
from jax.experimental.pallas import tpu as pltpu
from jax.experimental.pallas import tpu_sc as plsc

N = 10000
E = 320000
D = 128
DOUT = 40
D2 = 64
NPAD = 10240
NC = 2
NS = 16
CHUNK = 80
ROWS_PER_TILE = NPAD // NS

_MESH = dict(core_axis_name="c", subcore_axis_name="s")


def _sc_degrees(edges64):
    nblk = E // (NC * NS * CHUNK)

    @functools.partial(
        pl.kernel,
        out_type=jax.ShapeDtypeStruct((NC * NPAD,), jnp.float32),
        mesh=plsc.VectorSubcoreMesh(**_MESH),
        scratch_types=[
            pltpu.VMEM((2, nblk, CHUNK), jnp.int32),
            pltpu.VMEM((CHUNK,), jnp.float32),
            pltpu.VMEM((ROWS_PER_TILE,), jnp.float32),
            pltpu.SemaphoreType.DMA,
            pltpu.VMEM_SHARED((NPAD,), jnp.float32),
        ],
    )
    def deg_kernel(edges_hbm, out_hbm, idx_all, ones_v, zero_v, ssem, deg_s):
        c = lax.axis_index("c")
        s = lax.axis_index("s")
        w = c * NS + s

        def fill_zero(i, carry):
            zero_v[pl.ds(i * 16, 16)] = jnp.zeros((16,), jnp.float32)
            return carry

        lax.fori_loop(0, ROWS_PER_TILE // 16, fill_zero, 0)

        def fill_one(i, carry):
            ones_v[pl.ds(i * 16, 16)] = jnp.ones((16,), jnp.float32)
            return carry

        lax.fori_loop(0, CHUNK // 16, fill_one, 0)

        pltpu.sync_copy(edges_hbm.at[2 * w], idx_all.at[0])
        pltpu.sync_copy(edges_hbm.at[2 * w + 1], idx_all.at[1])
        pltpu.sync_copy(zero_v, deg_s.at[pl.ds(s * ROWS_PER_TILE, ROWS_PER_TILE)])
        plsc.subcore_barrier()

        for h in range(2):
            def step(j, carry):
                pltpu.async_copy(ones_v, deg_s.at[idx_all.at[h, j]], ssem, add=True)
                return carry

            lax.fori_loop(0, nblk, step, 0)

        def drain(j, carry):
            pltpu.make_async_copy(ones_v, deg_s.at[idx_all.at[0, 0]], ssem).wait()
            return carry

        lax.fori_loop(0, 2 * nblk, drain, 0)
        plsc.subcore_barrier()
        pltpu.sync_copy(
            deg_s.at[pl.ds(s * ROWS_PER_TILE, ROWS_PER_TILE)],
            out_hbm.at[pl.ds(c * NPAD + s * ROWS_PER_TILE, ROWS_PER_TILE)],
        )

    return deg_kernel(edges64)


def _sc_scatter(edges64, feat, dfeat):
    e_per_core = E // NC
    ept = e_per_core // NS
    csz = CHUNK
    NB = 3 if dfeat > 64 else 4
    nchunk = ept // csz
    NI = NB + 2
    ZR = 16

    @functools.partial(
        pl.kernel,
        out_type=jax.ShapeDtypeStruct((NC * NPAD, dfeat), jnp.float32),
        mesh=plsc.VectorSubcoreMesh(**_MESH),
        compiler_params=pltpu.CompilerParams(
            use_tc_tiling_on_sc=(dfeat % 128 == 0)
        ),
        scratch_types=[
            pltpu.VMEM((NI, csz), jnp.int32),
            pltpu.VMEM((NI, csz), jnp.int32),
            pltpu.VMEM((NB, csz, dfeat), jnp.float32),
            pltpu.VMEM((ZR, dfeat), jnp.float32),
            pltpu.SemaphoreType.DMA,
            pltpu.SemaphoreType.DMA,
            pltpu.SemaphoreType.DMA,
            pltpu.VMEM_SHARED((NPAD, dfeat), jnp.float32),
        ],
    )
    def scat_kernel(edges_hbm, feat_hbm, out_hbm, sidx, didx,
                    rows, zrows, isem, gsem, ssem, agg_s):
        c = lax.axis_index("c")
        s = lax.axis_index("s")
        wid = c * NS + s
        vpr = dfeat // 16

        def fill_zero(k, carry):
            zrows[k // vpr, pl.ds((k % vpr) * 16, 16)] = jnp.zeros((16,), jnp.float32)
            return carry

        lax.fori_loop(0, ZR * vpr, fill_zero, 0)

        def zero_chunk(k, carry):
            pltpu.sync_copy(
                zrows, agg_s.at[pl.ds(s * ROWS_PER_TILE + k * ZR, ZR)]
            )
            return carry

        lax.fori_loop(0, ROWS_PER_TILE // ZR, zero_chunk, 0)
        plsc.subcore_barrier()

        def load_idx(j, slot):
            pltpu.async_copy(edges_hbm.at[wid, j], sidx.at[slot], isem)
            pltpu.async_copy(edges_hbm.at[NC * NS + wid, j], didx.at[slot], isem)

        def wait_idx():
            pltpu.make_async_copy(edges_hbm.at[0, 0], sidx.at[0], isem).wait()
            pltpu.make_async_copy(edges_hbm.at[0, 0], didx.at[0], isem).wait()

        def gath(b, slot):
            pltpu.async_copy(feat_hbm.at[sidx.at[slot]], rows.at[b], gsem)

        def wait_gath():
            pltpu.make_async_copy(
                feat_hbm.at[sidx.at[0]], rows.at[0], gsem
            ).wait()

        def scat(b, slot):
            pltpu.async_copy(rows.at[b], agg_s.at[didx.at[slot]], ssem, add=True)

        def wait_scat():
            pltpu.make_async_copy(
                rows.at[0], agg_s.at[didx.at[0]], ssem
            ).wait()

        n = nchunk
        for j in range(NI):
            load_idx(j, j)
        for j in range(NB - 1):
            wait_idx()
            gath(j, j)
        wait_gath()
        scat(0, 0)
        wait_idx()
        gath((NB - 1) % NB, (NB - 1) % NI)
        wait_gath()
        scat(1 % NB, 1 % NI)

        def step(k, carry):
            wait_scat()
            load_idx(k + NI - 2, lax.rem(k + NI - 2, NI))
            wait_idx()
            gath(lax.rem(k + NB - 2, NB), lax.rem(k + NB - 2, NI))
            wait_gath()
            scat(lax.rem(k, NB), lax.rem(k, NI))
            return carry

        lax.fori_loop(2, n - NI + 2, step, 0)

        for k in range(n - NI + 2, n - NB + 2):
            wait_scat()
            wait_idx()
            gath((k + NB - 2) % NB, (k + NB - 2) % NI)
            wait_gath()
            scat(k % NB, k % NI)
        for k in range(n - NB + 2, n):
            wait_scat()
            wait_gath()
            scat(k % NB, k % NI)
        wait_scat()
        wait_scat()

        plsc.subcore_barrier()
        pltpu.sync_copy(
            agg_s.at[pl.ds(s * ROWS_PER_TILE, ROWS_PER_TILE)],
            out_hbm.at[pl.ds(c * NPAD + s * ROWS_PER_TILE, ROWS_PER_TILE)],
        )

    return scat_kernel(edges64, feat)


_TCR = 2000


def _norm_body(x_ref, do_ref, di_ref, xs_ref, ns_ref, nd_ref):
    ns = lax.rsqrt(do_ref[...] + 1.0)
    nd = lax.rsqrt(di_ref[...] + 1.0)
    xs_ref[...] = x_ref[...] * ns
    ns_ref[...] = ns
    nd_ref[...] = nd


def _tc_norm(x, do_col, di_col):
    row = lambda i: (i, 0)
    return pl.pallas_call(
        _norm_body,
        grid=(N // _TCR,),
        in_specs=[
            pl.BlockSpec((_TCR, D), row),
            pl.BlockSpec((_TCR, 1), row),
            pl.BlockSpec((_TCR, 1), row),
        ],
        out_specs=[
            pl.BlockSpec((_TCR, D), row),
            pl.BlockSpec((_TCR, 1), row),
            pl.BlockSpec((_TCR, 1), row),
        ],
        out_shape=[
            jax.ShapeDtypeStruct((N, D), jnp.float32),
            jax.ShapeDtypeStruct((N, 1), jnp.float32),
            jax.ShapeDtypeStruct((N, 1), jnp.float32),
        ],
    )(x, do_col, di_col)


def _mm_body(a0_ref, a1_ref, xs_ref, nd_ref, ns_ref, w1_ref, b1_ref, w2_ref, g2_ref):
    a = (a0_ref[...] + a1_ref[...] + xs_ref[...]) * nd_ref[...]
    h1 = (
        jnp.dot(a, w1_ref[...], preferred_element_type=jnp.float32,
                precision=lax.Precision.HIGHEST)
        + b1_ref[...]
    )
    g2_ref[...] = jnp.dot(
        h1 * ns_ref[...], w2_ref[...], preferred_element_type=jnp.float32,
        precision=lax.Precision.HIGHEST,
    )


def _tc_matmuls(a0, a1, xs, nd_col, ns_col, W1, b1r, W2p):
    row = lambda i: (i, 0)
    full = lambda i: (0, 0)
    return pl.pallas_call(
        _mm_body,
        grid=(N // _TCR,),
        in_specs=[
            pl.BlockSpec((_TCR, D), row),
            pl.BlockSpec((_TCR, D), row),
            pl.BlockSpec((_TCR, D), row),
            pl.BlockSpec((_TCR, 1), row),
            pl.BlockSpec((_TCR, 1), row),
            pl.BlockSpec((D, D), full),
            pl.BlockSpec((1, D), full),
            pl.BlockSpec((D, D2), full),
        ],
        out_specs=pl.BlockSpec((_TCR, D2), row),
        out_shape=jax.ShapeDtypeStruct((N, D2), jnp.float32),
    )(a0, a1, xs, nd_col, ns_col, W1, b1r, W2p)


def _final_body(a0_ref, a1_ref, g2_ref, nd_ref, b2_ref, out_ref):
    out_ref[...] = (
        (a0_ref[...] + a1_ref[...] + g2_ref[...]) * nd_ref[...] + b2_ref[...]
    )


def _tc_final(a0, a1, g2, nd_col, b2r):
    row = lambda i: (i, 0)
    full = lambda i: (0, 0)
    return pl.pallas_call(
        _final_body,
        grid=(N // _TCR,),
        in_specs=[
            pl.BlockSpec((_TCR, D2), row),
            pl.BlockSpec((_TCR, D2), row),
            pl.BlockSpec((_TCR, D2), row),
            pl.BlockSpec((_TCR, 1), row),
            pl.BlockSpec((1, D2), full),
        ],
        out_specs=pl.BlockSpec((_TCR, D2), row),
        out_shape=jax.ShapeDtypeStruct((N, D2), jnp.float32),
    )(a0, a1, g2, nd_col, b2r)


def kernel(x, edge_index, W1, b1, W2, b2):
    nblk = E // (NC * NS * CHUNK)
    edges64 = edge_index.reshape(2 * NC * NS, nblk, CHUNK)

    deg = _sc_degrees(edges64)
    do_col = deg[:N, None]
    di_col = deg[NPAD:NPAD + N, None]

    xs, ns_col, nd_col = _tc_norm(x, do_col, di_col)

    agg = _sc_scatter(edges64, xs, D)
    g2 = _tc_matmuls(
        agg[:N], agg[NPAD:NPAD + N], xs, nd_col, ns_col,
        W1, b1[None, :], jnp.pad(W2, ((0, 0), (0, D2 - DOUT))),
    )

    agg2 = _sc_scatter(edges64, g2, D2)
    out = _tc_final(
        agg2[:N], agg2[NPAD:NPAD + N], g2, nd_col,
        jnp.pad(b2, (0, D2 - DOUT))[None, :],
    )
    return out[:, :DOUT]
